# baseline (speedup 1.0000x reference)
import functools

import jax
import jax.numpy as jnp
from jax import lax
from jax.experimental import pallas as pl
from jax.experimental.pallas import tpu as pltpu
from jax.experimental.pallas import tpu_sc as plsc

NC = 2
NS = 16
NW = NC * NS
LANES = 16

CHUNK = 128
N_NODES = 10000
D_FEAT = 128

E_EDGES = 320000
CHUNKS_PER_W = 80
EDGES_PER_W = CHUNKS_PER_W * CHUNK
E_PAD = NW * EDGES_PER_W
DUMMY_ROW = N_NODES
N_PAD = 10240
ACC_ROWS = N_PAD
RPT = ACC_ROWS // NS


def _make_sc_segsum(with_count: bool):
  mesh = plsc.VectorSubcoreMesh(core_axis_name="c", subcore_axis_name="s")

  out_type = [jax.ShapeDtypeStruct((NC, ACC_ROWS, D_FEAT), jnp.float32)]
  scratch = [
      pltpu.VMEM_SHARED((ACC_ROWS, D_FEAT), jnp.float32),
      pltpu.VMEM((2, CHUNK), jnp.int32),
      pltpu.VMEM((CHUNKS_PER_W, CHUNK), jnp.int32),
      pltpu.VMEM((CHUNK, D_FEAT), jnp.float32),
      pltpu.VMEM((CHUNK, D_FEAT), jnp.float32),
      pltpu.SemaphoreType.DMA,
      pltpu.SemaphoreType.DMA,
      pltpu.SemaphoreType.DMA,
  ]
  if with_count:
    out_type.append(jax.ShapeDtypeStruct((NC, ACC_ROWS), jnp.float32))
    scratch.append(pltpu.VMEM_SHARED((ACC_ROWS,), jnp.float32))
    scratch.append(pltpu.VMEM((CHUNK,), jnp.float32))

  def body(table_hbm, src_hbm, dst_hbm, z2_hbm, z1_hbm, *rest):
    if with_count:
      (agg_out, cnt_out, acc_sh, src_db, dst_all, rows0, rows1,
       sem_g0, sem_g1, sem_c, cnt_sh, ones_v) = rest
    else:
      (agg_out, acc_sh, src_db, dst_all, rows0, rows1,
       sem_g0, sem_g1, sem_c) = rest
    c = lax.axis_index("c")
    s = lax.axis_index("s")
    wid = s * NC + c
    cbase = wid * CHUNKS_PER_W

    pltpu.sync_copy(dst_hbm.at[pl.ds(cbase, CHUNKS_PER_W)], dst_all)
    pltpu.sync_copy(z2_hbm.at[pl.ds(s * RPT, RPT)], acc_sh.at[pl.ds(s * RPT, RPT)])
    if with_count:
      pltpu.sync_copy(z1_hbm.at[pl.ds(s * RPT, RPT)], cnt_sh.at[pl.ds(s * RPT, RPT)])
      ones16 = jnp.ones((LANES,), jnp.float32)
      for k in range(CHUNK // LANES):
        ones_v[pl.ds(k * LANES, LANES)] = ones16
    plsc.subcore_barrier()

    n_pair = CHUNKS_PER_W // 2
    pltpu.sync_copy(src_hbm.at[cbase], src_db.at[0])
    pltpu.async_copy(table_hbm.at[src_db.at[0]], rows0, sem_g0)
    pltpu.sync_copy(src_hbm.at[cbase + 1], src_db.at[1])

    def pair_body(p, carry):
      j0 = 2 * p
      j1 = j0 + 1
      pltpu.async_copy(table_hbm.at[src_db.at[1]], rows1, sem_g1)
      if with_count:
        pltpu.async_copy(ones_v, cnt_sh.at[dst_all.at[j0]], sem_c, add=True)
        pltpu.async_copy(ones_v, cnt_sh.at[dst_all.at[j1]], sem_c, add=True)
      pltpu.make_async_copy(table_hbm.at[src_db.at[0]], rows0, sem_g0).wait()
      pltpu.sync_copy(rows0, acc_sh.at[dst_all.at[j0]], add=True)

      @pl.when(p < n_pair - 1)
      def _():
        pltpu.sync_copy(src_hbm.at[cbase + j0 + 2], src_db.at[0])
        pltpu.async_copy(table_hbm.at[src_db.at[0]], rows0, sem_g0)

      pltpu.make_async_copy(table_hbm.at[src_db.at[1]], rows1, sem_g1).wait()
      pltpu.sync_copy(rows1, acc_sh.at[dst_all.at[j1]], add=True)

      @pl.when(p < n_pair - 1)
      def _():
        pltpu.sync_copy(src_hbm.at[cbase + j1 + 2], src_db.at[1])
      if with_count:
        pltpu.make_async_copy(ones_v, cnt_sh.at[dst_all.at[j0]], sem_c).wait()
        pltpu.make_async_copy(ones_v, cnt_sh.at[dst_all.at[j1]], sem_c).wait()
      return carry

    lax.fori_loop(0, n_pair, pair_body, 0)
    plsc.subcore_barrier()

    pltpu.sync_copy(acc_sh.at[pl.ds(s * RPT, RPT)],
                    agg_out.at[c, pl.ds(s * RPT, RPT)])
    if with_count:
      pltpu.sync_copy(cnt_sh.at[pl.ds(s * RPT, RPT)],
                      cnt_out.at[c, pl.ds(s * RPT, RPT)])

  return pl.kernel(body, out_type=tuple(out_type), mesh=mesh,
                   scratch_types=scratch)


_sc_segsum_count = _make_sc_segsum(True)
_sc_segsum = _make_sc_segsum(False)


def _layer1_body(agg_ref, cnt_ref, x_ref, wl_ref, wr_ref, b_ref, w2l_ref,
                 h_ref, p_ref):
  cnt = jnp.maximum(jnp.sum(cnt_ref[...], axis=0), 1.0)
  agg = agg_ref[0] + agg_ref[1]
  mean = agg / cnt[:, None]
  h = jnp.tanh(
      jnp.dot(mean, wl_ref[...], preferred_element_type=jnp.float32)
      + jnp.dot(x_ref[...], wr_ref[...], preferred_element_type=jnp.float32)
      + b_ref[...])
  h_ref[...] = h
  p_ref[...] = jnp.dot(h, w2l_ref[...], preferred_element_type=jnp.float32)


def _layer2_body(agg_ref, cnt_ref, h_ref, wr_ref, b_ref, out_ref):
  cnt = jnp.maximum(jnp.sum(cnt_ref[...], axis=0), 1.0)
  mean_l = (agg_ref[0] + agg_ref[1]) / cnt[:, None]
  out_ref[...] = jnp.tanh(
      mean_l
      + jnp.dot(h_ref[...], wr_ref[...], preferred_element_type=jnp.float32)
      + b_ref[...])


_BLK = 1024


def kernel(x, edge_index, W1_l, W1_r, b1, W2_l, W2_r, b2):
  src = edge_index[0].astype(jnp.int32)
  dst = edge_index[1].astype(jnp.int32)
  pad = E_PAD - src.shape[0]
  src_p = jnp.concatenate([src, jnp.zeros((pad,), jnp.int32)]).reshape(-1, CHUNK)
  dst_p = jnp.concatenate([dst, jnp.full((pad,), DUMMY_ROW, jnp.int32)]).reshape(-1, CHUNK)
  z2 = jnp.zeros((ACC_ROWS, D_FEAT), jnp.float32)
  z1 = jnp.zeros((ACC_ROWS,), jnp.float32)

  n = x.shape[0]
  x_pad = jnp.pad(x, ((0, N_PAD - n), (0, 0)))

  agg1, cnt_part = _sc_segsum_count(x_pad, src_p, dst_p, z2, z1)

  grid = N_PAD // _BLK
  h, p = pl.pallas_call(
      _layer1_body,
      grid=(grid,),
      in_specs=[
          pl.BlockSpec((NC, _BLK, D_FEAT), lambda i: (0, i, 0)),
          pl.BlockSpec((NC, _BLK), lambda i: (0, i)),
          pl.BlockSpec((_BLK, D_FEAT), lambda i: (i, 0)),
          pl.BlockSpec((D_FEAT, 256), lambda i: (0, 0)),
          pl.BlockSpec((D_FEAT, 256), lambda i: (0, 0)),
          pl.BlockSpec((1, 256), lambda i: (0, 0)),
          pl.BlockSpec((256, D_FEAT), lambda i: (0, 0)),
      ],
      out_specs=[
          pl.BlockSpec((_BLK, 256), lambda i: (i, 0)),
          pl.BlockSpec((_BLK, D_FEAT), lambda i: (i, 0)),
      ],
      out_shape=[
          jax.ShapeDtypeStruct((N_PAD, 256), jnp.float32),
          jax.ShapeDtypeStruct((N_PAD, D_FEAT), jnp.float32),
      ],
  )(agg1, cnt_part, x_pad, W1_l, W1_r, b1.reshape(1, 256), W2_l)

  (agg2,) = _sc_segsum(p, src_p, dst_p, z2, z1)

  out = pl.pallas_call(
      _layer2_body,
      grid=(grid,),
      in_specs=[
          pl.BlockSpec((NC, _BLK, D_FEAT), lambda i: (0, i, 0)),
          pl.BlockSpec((NC, _BLK), lambda i: (0, i)),
          pl.BlockSpec((_BLK, 256), lambda i: (i, 0)),
          pl.BlockSpec((256, D_FEAT), lambda i: (0, 0)),
          pl.BlockSpec((1, D_FEAT), lambda i: (0, 0)),
      ],
      out_specs=pl.BlockSpec((_BLK, D_FEAT), lambda i: (i, 0)),
      out_shape=jax.ShapeDtypeStruct((N_PAD, D_FEAT), jnp.float32),
  )(agg2, cnt_part, h, W2_r, b2.reshape(1, D_FEAT))

  return out[:n]


# speedup vs baseline: 2.4961x; 2.4961x over previous
import jax
import jax.numpy as jnp
from jax import lax
from jax.experimental import pallas as pl
from jax.experimental.pallas import tpu as pltpu
from jax.experimental.pallas import tpu_sc as plsc

NC = 2
NS = 16
NW = NC * NS
LANES = 16

N_NODES = 10000
E_EDGES = 320000
D_FEAT = 128

CHUNK = 128
NBUF = 2
N_CHUNKS = E_EDGES // CHUNK
BASE_CHUNKS = N_CHUNKS // NW
EXTRA_CHUNKS = N_CHUNKS - BASE_CHUNKS * NW

ACC_ROWS = 10240
RPT = ACC_ROWS // NS


def _make_sc_segsum(with_count: bool):
  mesh = plsc.VectorSubcoreMesh(core_axis_name="c", subcore_axis_name="s")

  out_type = [jax.ShapeDtypeStruct((NC, ACC_ROWS, D_FEAT), jnp.float32)]
  scratch = [
      pltpu.VMEM_SHARED((ACC_ROWS, D_FEAT), jnp.float32),
  ] + [pltpu.VMEM((CHUNK,), jnp.int32) for _ in range(2 * NBUF)
  ] + [pltpu.VMEM((CHUNK, D_FEAT), jnp.float32) for _ in range(NBUF)
  ] + [pltpu.SemaphoreType.DMA for _ in range(NBUF)] + [
      pltpu.SemaphoreType.DMA
  ]
  if with_count:
    out_type.append(jax.ShapeDtypeStruct((NC, ACC_ROWS), jnp.float32))
    scratch.append(pltpu.VMEM_SHARED((ACC_ROWS,), jnp.float32))
    scratch.append(pltpu.VMEM((CHUNK,), jnp.float32))

  def body(table_hbm, ei_hbm, z2_hbm, z1_hbm, *rest):
    if with_count:
      (agg_out, cnt_out, acc_sh, *rb) = rest
    else:
      (agg_out, acc_sh, *rb) = rest
    src_b = rb[:NBUF]
    dst_b = rb[NBUF:2 * NBUF]
    rows = rb[2 * NBUF:3 * NBUF]
    sem_g = rb[3 * NBUF:4 * NBUF]
    sem_c = rb[4 * NBUF]
    if with_count:
      cnt_sh, ones_v = rb[4 * NBUF + 1:]
    c = lax.axis_index("c")
    s = lax.axis_index("s")
    wid = s * NC + c
    ebase = wid * (BASE_CHUNKS * CHUNK)

    pltpu.sync_copy(z2_hbm, acc_sh.at[pl.ds(s * RPT, RPT)])
    if with_count:
      pltpu.sync_copy(z1_hbm, cnt_sh.at[pl.ds(s * RPT, RPT)])
      ones16 = jnp.ones((LANES,), jnp.float32)
      for k in range(CHUNK // LANES):
        ones_v[pl.ds(k * LANES, LANES)] = ones16
    plsc.subcore_barrier()

    n_group = BASE_CHUNKS // NBUF
    for b in range(NBUF):
      pltpu.sync_copy(
          ei_hbm.at[pl.ds(pl.multiple_of(ebase + b * CHUNK, 128), CHUNK)],
          src_b[b])
      pltpu.sync_copy(
          ei_hbm.at[pl.ds(pl.multiple_of(E_EDGES + ebase + b * CHUNK, 128),
                          CHUNK)],
          dst_b[b])
      pltpu.async_copy(table_hbm.at[src_b[b]], rows[b], sem_g[b])

    def group_body(g, carry):
      for b in range(NBUF):
        j = g * NBUF + b
        if with_count:
          pltpu.async_copy(ones_v, cnt_sh.at[dst_b[b]], sem_c, add=True)
        pltpu.make_async_copy(table_hbm.at[src_b[b]], rows[b],
                              sem_g[b]).wait()
        pltpu.sync_copy(rows[b], acc_sh.at[dst_b[b]], add=True)
        if with_count:
          pltpu.make_async_copy(ones_v, cnt_sh.at[dst_b[b]], sem_c).wait()

        @pl.when(g < n_group - 1)
        def _():
          off = pl.multiple_of(ebase + (j + NBUF) * CHUNK, 128)
          pltpu.sync_copy(ei_hbm.at[pl.ds(off, CHUNK)], src_b[b])
          pltpu.sync_copy(
              ei_hbm.at[pl.ds(pl.multiple_of(E_EDGES + off, 128), CHUNK)],
              dst_b[b])
          pltpu.async_copy(table_hbm.at[src_b[b]], rows[b], sem_g[b])
      return carry

    lax.fori_loop(0, n_group, group_body, 0)

    @pl.when(wid < EXTRA_CHUNKS)
    def _():
      xoff = pl.multiple_of((BASE_CHUNKS * NW + wid) * CHUNK, 128)
      pltpu.sync_copy(ei_hbm.at[pl.ds(xoff, CHUNK)], src_b[0])
      pltpu.sync_copy(
          ei_hbm.at[pl.ds(pl.multiple_of(E_EDGES + xoff, 128), CHUNK)],
          dst_b[0])
      pltpu.async_copy(table_hbm.at[src_b[0]], rows[0], sem_g[0]).wait()
      pltpu.sync_copy(rows[0], acc_sh.at[dst_b[0]], add=True)
      if with_count:
        pltpu.sync_copy(ones_v, cnt_sh.at[dst_b[0]], add=True)
    plsc.subcore_barrier()

    pltpu.sync_copy(acc_sh.at[pl.ds(s * RPT, RPT)],
                    agg_out.at[c, pl.ds(s * RPT, RPT)])
    if with_count:
      pltpu.sync_copy(cnt_sh.at[pl.ds(s * RPT, RPT)],
                      cnt_out.at[c, pl.ds(s * RPT, RPT)])

  return pl.kernel(body, out_type=tuple(out_type), mesh=mesh,
                   scratch_types=scratch)


_sc_segsum_count = _make_sc_segsum(True)
_sc_segsum = _make_sc_segsum(False)


def _xr_body(x_ref, wr_ref, b_ref, xr_ref):
  xr_ref[...] = (
      jnp.dot(x_ref[...], wr_ref[...], preferred_element_type=jnp.float32)
      + b_ref[...])


def _layer1_body(agg_ref, cnt_ref, xr_ref, wl_ref, w2l_ref, w2r_ref, b2_ref,
                 p_ref, hr_ref):
  cnt = jnp.maximum(jnp.sum(cnt_ref[...], axis=1), 1.0)
  agg = agg_ref[0] + agg_ref[1]
  mean = agg / cnt[:, None]
  h = jnp.tanh(
      jnp.dot(mean, wl_ref[...], preferred_element_type=jnp.float32)
      + xr_ref[...])
  p_ref[...] = jnp.dot(h, w2l_ref[...], preferred_element_type=jnp.float32)
  hr_ref[...] = (
      jnp.dot(h, w2r_ref[...], preferred_element_type=jnp.float32)
      + b2_ref[...])


def _layer2_body(agg_ref, cnt_ref, hr_ref, out_ref):
  cnt = jnp.maximum(jnp.sum(cnt_ref[...], axis=1), 1.0)
  mean_l = (agg_ref[0] + agg_ref[1]) / cnt[:, None]
  out_ref[...] = jnp.tanh(mean_l + hr_ref[...])


_BLK = 1000


def kernel(x, edge_index, W1_l, W1_r, b1, W2_l, W2_r, b2):
  ei = edge_index.astype(jnp.int32).reshape(-1)
  z2 = jnp.zeros((RPT, D_FEAT), jnp.float32)
  z1 = jnp.zeros((RPT,), jnp.float32)
  n = x.shape[0]

  agg1, cnt_part = _sc_segsum_count(x, ei, z2, z1)
  cnt_t = cnt_part.T

  grid = n // _BLK
  xr = pl.pallas_call(
      _xr_body,
      grid=(grid,),
      in_specs=[
          pl.BlockSpec((_BLK, D_FEAT), lambda i: (i, 0)),
          pl.BlockSpec((D_FEAT, 256), lambda i: (0, 0)),
          pl.BlockSpec((1, 256), lambda i: (0, 0)),
      ],
      out_specs=pl.BlockSpec((_BLK, 256), lambda i: (i, 0)),
      out_shape=jax.ShapeDtypeStruct((n, 256), jnp.float32),
  )(x, W1_r, b1.reshape(1, 256))

  p, hr = pl.pallas_call(
      _layer1_body,
      grid=(grid,),
      in_specs=[
          pl.BlockSpec((NC, _BLK, D_FEAT), lambda i: (0, i, 0)),
          pl.BlockSpec((_BLK, NC), lambda i: (i, 0)),
          pl.BlockSpec((_BLK, 256), lambda i: (i, 0)),
          pl.BlockSpec((D_FEAT, 256), lambda i: (0, 0)),
          pl.BlockSpec((256, D_FEAT), lambda i: (0, 0)),
          pl.BlockSpec((256, D_FEAT), lambda i: (0, 0)),
          pl.BlockSpec((1, D_FEAT), lambda i: (0, 0)),
      ],
      out_specs=[
          pl.BlockSpec((_BLK, D_FEAT), lambda i: (i, 0)),
          pl.BlockSpec((_BLK, D_FEAT), lambda i: (i, 0)),
      ],
      out_shape=[
          jax.ShapeDtypeStruct((n, D_FEAT), jnp.float32),
          jax.ShapeDtypeStruct((n, D_FEAT), jnp.float32),
      ],
  )(agg1, cnt_t, xr, W1_l, W2_l, W2_r, b2.reshape(1, D_FEAT))

  (agg2,) = _sc_segsum(p, ei, z2, z1)

  out = pl.pallas_call(
      _layer2_body,
      grid=(grid,),
      in_specs=[
          pl.BlockSpec((NC, _BLK, D_FEAT), lambda i: (0, i, 0)),
          pl.BlockSpec((_BLK, NC), lambda i: (i, 0)),
          pl.BlockSpec((_BLK, D_FEAT), lambda i: (i, 0)),
      ],
      out_specs=pl.BlockSpec((_BLK, D_FEAT), lambda i: (i, 0)),
      out_shape=jax.ShapeDtypeStruct((n, D_FEAT), jnp.float32),
  )(agg2, cnt_t, hr)

  return out


# speedup vs baseline: 2.8379x; 1.1369x over previous
import jax
import jax.numpy as jnp
from jax import lax
from jax.experimental import pallas as pl
from jax.experimental.pallas import tpu as pltpu
from jax.experimental.pallas import tpu_sc as plsc

NC = 2
NS = 16
NW = NC * NS
LANES = 16

CHUNK = 128
NBUF = 2
N_NODES = 10000
D_FEAT = 128

E_EDGES = 320000
CHUNKS_PER_W = 80
EDGES_PER_W = CHUNKS_PER_W * CHUNK
E_PAD = NW * EDGES_PER_W
DUMMY_ROW = N_NODES
N_PAD = 10240
ACC_ROWS = N_PAD
RPT = ACC_ROWS // NS


def _make_sc_segsum(with_count: bool):
  mesh = plsc.VectorSubcoreMesh(core_axis_name="c", subcore_axis_name="s")

  out_type = [jax.ShapeDtypeStruct((NC, ACC_ROWS, D_FEAT), jnp.float32)]
  scratch = [
      pltpu.VMEM_SHARED((ACC_ROWS, D_FEAT), jnp.float32),
      pltpu.VMEM((NBUF, 2, CHUNK), jnp.int32),
  ] + [pltpu.VMEM((CHUNK, D_FEAT), jnp.float32) for _ in range(NBUF)] + [
      pltpu.SemaphoreType.DMA for _ in range(NBUF)
  ] + [pltpu.SemaphoreType.DMA]
  if with_count:
    out_type.append(jax.ShapeDtypeStruct((NC, ACC_ROWS), jnp.float32))
    scratch.append(pltpu.VMEM_SHARED((ACC_ROWS,), jnp.float32))
    scratch.append(pltpu.VMEM((CHUNK,), jnp.float32))

  def body(table_hbm, idx_hbm, z2_hbm, z1_hbm, *rest):
    if with_count:
      (agg_out, cnt_out, acc_sh, idx_db, *rb) = rest
      rows = rb[:NBUF]
      sem_g = rb[NBUF:2 * NBUF]
      sem_c = rb[2 * NBUF]
      cnt_sh, ones_v = rb[2 * NBUF + 1:]
    else:
      (agg_out, acc_sh, idx_db, *rb) = rest
      rows = rb[:NBUF]
      sem_g = rb[NBUF:2 * NBUF]
      sem_c = rb[2 * NBUF]
    c = lax.axis_index("c")
    s = lax.axis_index("s")
    wid = s * NC + c
    cbase = wid * CHUNKS_PER_W

    pltpu.sync_copy(z2_hbm.at[pl.ds(s * RPT, RPT)], acc_sh.at[pl.ds(s * RPT, RPT)])
    if with_count:
      pltpu.sync_copy(z1_hbm.at[pl.ds(s * RPT, RPT)], cnt_sh.at[pl.ds(s * RPT, RPT)])
      ones16 = jnp.ones((LANES,), jnp.float32)
      for k in range(CHUNK // LANES):
        ones_v[pl.ds(k * LANES, LANES)] = ones16
    plsc.subcore_barrier()

    n_group = CHUNKS_PER_W // NBUF
    for b in range(NBUF):
      pltpu.sync_copy(idx_hbm.at[cbase + b], idx_db.at[b])
      pltpu.async_copy(table_hbm.at[idx_db.at[b, 0]], rows[b], sem_g[b])

    def group_body(g, carry):
      for b in range(NBUF):
        j = g * NBUF + b
        if with_count:
          pltpu.async_copy(ones_v, cnt_sh.at[idx_db.at[b, 1]], sem_c, add=True)
        pltpu.make_async_copy(table_hbm.at[idx_db.at[b, 0]], rows[b],
                              sem_g[b]).wait()
        pltpu.sync_copy(rows[b], acc_sh.at[idx_db.at[b, 1]], add=True)
        if with_count:
          pltpu.make_async_copy(ones_v, cnt_sh.at[idx_db.at[b, 1]], sem_c).wait()

        @pl.when(g < n_group - 1)
        def _():
          pltpu.sync_copy(idx_hbm.at[cbase + j + NBUF], idx_db.at[b])
          pltpu.async_copy(table_hbm.at[idx_db.at[b, 0]], rows[b], sem_g[b])
      return carry

    lax.fori_loop(0, n_group, group_body, 0)
    plsc.subcore_barrier()

    pltpu.sync_copy(acc_sh.at[pl.ds(s * RPT, RPT)],
                    agg_out.at[c, pl.ds(s * RPT, RPT)])
    if with_count:
      pltpu.sync_copy(cnt_sh.at[pl.ds(s * RPT, RPT)],
                      cnt_out.at[c, pl.ds(s * RPT, RPT)])

  return pl.kernel(body, out_type=tuple(out_type), mesh=mesh,
                   scratch_types=scratch)


_sc_segsum_count = _make_sc_segsum(True)
_sc_segsum = _make_sc_segsum(False)


def _xr_body(x_ref, wr_ref, b_ref, xr_ref):
  xr_ref[...] = (
      jnp.dot(x_ref[...], wr_ref[...], preferred_element_type=jnp.float32)
      + b_ref[...])


def _layer1_body(agg_ref, cnt_ref, xr_ref, wl_ref, w2l_ref, w2r_ref, b2_ref,
                 p_ref, hr_ref):
  cnt = jnp.maximum(jnp.sum(cnt_ref[...], axis=0), 1.0)
  agg = agg_ref[0] + agg_ref[1]
  mean = agg / cnt[:, None]
  h = jnp.tanh(
      jnp.dot(mean, wl_ref[...], preferred_element_type=jnp.float32)
      + xr_ref[...])
  p_ref[...] = jnp.dot(h, w2l_ref[...], preferred_element_type=jnp.float32)
  hr_ref[...] = (
      jnp.dot(h, w2r_ref[...], preferred_element_type=jnp.float32)
      + b2_ref[...])


def _layer2_body(agg_ref, cnt_ref, hr_ref, out_ref):
  cnt = jnp.maximum(jnp.sum(cnt_ref[...], axis=0), 1.0)
  mean_l = (agg_ref[0] + agg_ref[1]) / cnt[:, None]
  out_ref[...] = jnp.tanh(mean_l + hr_ref[...])


_BLK = 1024


def kernel(x, edge_index, W1_l, W1_r, b1, W2_l, W2_r, b2):
  src = edge_index[0].astype(jnp.int32)
  dst = edge_index[1].astype(jnp.int32)
  pad = E_PAD - src.shape[0]
  ar = jnp.arange(pad, dtype=jnp.int32)
  dummy_dst = DUMMY_ROW + ar % (ACC_ROWS - DUMMY_ROW)
  dummy_src = ar % N_NODES
  src_p = jnp.concatenate([src, dummy_src]).reshape(-1, CHUNK)
  dst_p = jnp.concatenate([dst, dummy_dst]).reshape(-1, CHUNK)
  idx2 = jnp.stack([src_p, dst_p], axis=1)
  z2 = jnp.zeros((ACC_ROWS, D_FEAT), jnp.float32)
  z1 = jnp.zeros((ACC_ROWS,), jnp.float32)

  n = x.shape[0]
  x_pad = jnp.pad(x, ((0, N_PAD - n), (0, 0)))

  agg1, cnt_part = _sc_segsum_count(x_pad, idx2, z2, z1)

  grid = N_PAD // _BLK
  xr = pl.pallas_call(
      _xr_body,
      grid=(grid,),
      in_specs=[
          pl.BlockSpec((_BLK, D_FEAT), lambda i: (i, 0)),
          pl.BlockSpec((D_FEAT, 256), lambda i: (0, 0)),
          pl.BlockSpec((1, 256), lambda i: (0, 0)),
      ],
      out_specs=pl.BlockSpec((_BLK, 256), lambda i: (i, 0)),
      out_shape=jax.ShapeDtypeStruct((N_PAD, 256), jnp.float32),
  )(x_pad, W1_r, b1.reshape(1, 256))

  p, hr = pl.pallas_call(
      _layer1_body,
      grid=(grid,),
      in_specs=[
          pl.BlockSpec((NC, _BLK, D_FEAT), lambda i: (0, i, 0)),
          pl.BlockSpec((NC, _BLK), lambda i: (0, i)),
          pl.BlockSpec((_BLK, 256), lambda i: (i, 0)),
          pl.BlockSpec((D_FEAT, 256), lambda i: (0, 0)),
          pl.BlockSpec((256, D_FEAT), lambda i: (0, 0)),
          pl.BlockSpec((256, D_FEAT), lambda i: (0, 0)),
          pl.BlockSpec((1, D_FEAT), lambda i: (0, 0)),
      ],
      out_specs=[
          pl.BlockSpec((_BLK, D_FEAT), lambda i: (i, 0)),
          pl.BlockSpec((_BLK, D_FEAT), lambda i: (i, 0)),
      ],
      out_shape=[
          jax.ShapeDtypeStruct((N_PAD, D_FEAT), jnp.float32),
          jax.ShapeDtypeStruct((N_PAD, D_FEAT), jnp.float32),
      ],
  )(agg1, cnt_part, xr, W1_l, W2_l, W2_r, b2.reshape(1, D_FEAT))

  (agg2,) = _sc_segsum(p, idx2, z2, z1)

  out = pl.pallas_call(
      _layer2_body,
      grid=(grid,),
      in_specs=[
          pl.BlockSpec((NC, _BLK, D_FEAT), lambda i: (0, i, 0)),
          pl.BlockSpec((NC, _BLK), lambda i: (0, i)),
          pl.BlockSpec((_BLK, D_FEAT), lambda i: (i, 0)),
      ],
      out_specs=pl.BlockSpec((_BLK, D_FEAT), lambda i: (i, 0)),
      out_shape=jax.ShapeDtypeStruct((N_PAD, D_FEAT), jnp.float32),
  )(agg2, cnt_part, hr)

  return out[:n]
